# baseline (speedup 1.0000x reference)
import functools

import jax
import jax.numpy as jnp
from jax import lax
from jax.experimental import pallas as pl
from jax.experimental.pallas import tpu as pltpu
from jax.experimental.pallas import tpu_sc as plsc

NC = 2
NS = 16
NW = NC * NS


def _make_deg_hist(n, e, chunk, w, npad):
    epw = e // NW
    nchunk = epw // chunk
    rpt = npad // NS
    mesh = plsc.VectorSubcoreMesh(core_axis_name="c", subcore_axis_name="s",
                                  num_cores=NC)

    grp = 5

    @functools.partial(
        pl.kernel,
        out_type=jax.ShapeDtypeStruct((NC, npad, w), jnp.float32),
        mesh=mesh,
        scratch_types=[
            pltpu.VMEM((nchunk, chunk), jnp.int32),
            pltpu.VMEM((chunk, w), jnp.float32),
            pltpu.VMEM_SHARED((npad, w), jnp.float32),
            pltpu.SemaphoreType.DMA,
        ],
        compiler_params=pltpu.CompilerParams(use_tc_tiling_on_sc=False),
    )
    def deg_hist(dst_hbm, ones_hbm, zhist_hbm, out_hbm, dstv, rows, acc, sem):
        c = lax.axis_index("c")
        s = lax.axis_index("s")
        wid = s * NC + c
        pltpu.sync_copy(zhist_hbm, acc.at[pl.ds(s * rpt, rpt)])
        pltpu.sync_copy(ones_hbm, rows)
        pltpu.sync_copy(dst_hbm.at[wid], dstv)
        plsc.subcore_barrier()

        def body(j0, _):
            for b in range(grp):
                pltpu.async_copy(rows, acc.at[dstv.at[j0 * grp + b]], sem,
                                 add=True)
            for b in range(grp):
                pltpu.make_async_copy(rows, acc.at[dstv.at[0]], sem).wait()
            return _

        lax.fori_loop(0, nchunk // grp, body, 0)
        plsc.subcore_barrier()
        pltpu.sync_copy(acc.at[pl.ds(s * rpt, rpt)],
                        out_hbm.at[c, pl.ds(s * rpt, rpt)])

    return deg_hist


def _make_edge_agg(n, e, d, chunk, npad):
    epw = e // NW
    nchunk = epw // chunk
    rpt = npad // NS
    mesh = plsc.VectorSubcoreMesh(core_axis_name="c", subcore_axis_name="s",
                                  num_cores=NC)
    nbuf = 4
    nbi = 8
    assert nchunk >= 10

    @functools.partial(
        pl.kernel,
        out_type=jax.ShapeDtypeStruct((NC, npad, d), jnp.float32),
        mesh=mesh,
        scratch_types=[
            [pltpu.VMEM((chunk,), jnp.int32)] * nbi,
            [pltpu.VMEM((chunk,), jnp.int32)] * nbi,
            [pltpu.VMEM((chunk, d), jnp.float32)] * nbuf,
            pltpu.VMEM_SHARED((npad, d), jnp.float32),
            [pltpu.SemaphoreType.DMA] * nbi,
            [pltpu.SemaphoreType.DMA] * nbi,
            [pltpu.SemaphoreType.DMA] * nbuf,
            [pltpu.SemaphoreType.DMA] * nbuf,
        ],
    )
    def edge_agg(s_hbm, src_hbm, dst_hbm, ztile_hbm, out_hbm,
                 srcb, dstb, rows, acc, isem, jsem, gsem, ssem):
        c = lax.axis_index("c")
        s = lax.axis_index("s")
        wid = s * NC + c
        pltpu.sync_copy(ztile_hbm, acc.at[pl.ds(s * rpt, rpt)])
        plsc.subcore_barrier()

        base = wid * epw

        def idx_start(k, bi):
            off = base + k * chunk
            pltpu.async_copy(src_hbm.at[pl.ds(off, chunk)], srcb[bi],
                             isem[bi])
            pltpu.async_copy(dst_hbm.at[pl.ds(off, chunk)], dstb[bi],
                             jsem[bi])

        def idx_wait(bi):
            pltpu.make_async_copy(src_hbm.at[pl.ds(0, chunk)], srcb[bi],
                                  isem[bi]).wait()
            pltpu.make_async_copy(dst_hbm.at[pl.ds(0, chunk)], dstb[bi],
                                  jsem[bi]).wait()

        def g_start(bi, br):
            pltpu.async_copy(s_hbm.at[srcb[bi]], rows[br], gsem[br])

        def g_wait(br):
            pltpu.make_async_copy(s_hbm.at[srcb[0]], rows[br],
                                  gsem[br]).wait()

        def s_start(bij, br):
            pltpu.async_copy(rows[br], acc.at[dstb[bij]], ssem[br], add=True)

        def s_wait(br):
            pltpu.make_async_copy(rows[0], acc.at[dstb[0]], ssem[br]).wait()

        def step(j, jm4, jm8, do_sswait, do_gather, do_idx):
            g_wait(jm4)
            s_start(jm8, jm4)
            if do_sswait:
                s_wait((jm4 + 2) % nbuf)
            if do_gather:
                idx_wait((jm8 + 2) % nbi)
                g_start((jm8 + 2) % nbi, (jm4 + 2) % nbuf)
            if do_idx:
                idx_start(j + 4, (jm8 + 4) % nbi)

        for k in range(4):
            idx_start(k, k)
        idx_wait(0)
        g_start(0, 0)
        idx_wait(1)
        g_start(1, 1)
        step(0, 0, 0, False, True, True)
        step(1, 1, 1, False, True, True)

        main0 = 2
        nmain = ((min(nchunk - 4, nchunk - 2) - main0) // nbi) * nbi

        def body(j0, _):
            jb = main0 + j0 * nbi
            for i in range(nbi):
                step(jb + i, (main0 + i) % nbuf, (main0 + i) % nbi,
                     True, True, True)
            return _

        lax.fori_loop(0, nmain // nbi, body, 0)
        for j in range(main0 + nmain, nchunk):
            step(j, j % nbuf, j % nbi,
                 True, j + 2 < nchunk, j + 4 < nchunk)
        s_wait((nchunk - 2) % nbuf)
        s_wait((nchunk - 1) % nbuf)
        plsc.subcore_barrier()
        pltpu.sync_copy(acc.at[pl.ds(s * rpt, rpt)],
                        out_hbm.at[c, pl.ds(s * rpt, rpt)])

    return edge_agg


def _tc1_body(degp_ref, x_ref, w1_ref, dinv_ref, s1_ref):
    n = x_ref.shape[0]
    w = degp_ref.shape[2]
    degp = degp_ref[0, :n] + degp_ref[1, :n]
    ones = jnp.full((w, 1), 1.0 / w, jnp.float32)
    deg = jnp.dot(degp, ones, preferred_element_type=jnp.float32)
    dinv = jax.lax.rsqrt(deg + 1.0)
    dinv_ref[...] = dinv
    h = jnp.dot(x_ref[...], w1_ref[...], preferred_element_type=jnp.float32)
    s1_ref[...] = h * dinv


def _tc2_body(aggp_ref, s_ref, dinv_ref, b_ref, w2_ref, s2_ref):
    n = s_ref.shape[0]
    dinv = dinv_ref[...]
    agg = aggp_ref[0, :n] + aggp_ref[1, :n] + s_ref[...]
    h = jnp.maximum(agg * dinv + b_ref[...], 0.0)
    s2_ref[...] = jnp.dot(h, w2_ref[...],
                          preferred_element_type=jnp.float32) * dinv


def _tc3_body(aggp_ref, s_ref, dinv_ref, b_ref, batch_ref, wfc_ref, bfc_ref,
              out_ref, *, g):
    n = s_ref.shape[0]
    agg = aggp_ref[0, :n] + aggp_ref[1, :n] + s_ref[...]
    h = jnp.maximum(agg * dinv_ref[...] + b_ref[...], 0.0)
    gids = lax.broadcasted_iota(jnp.int32, (n, g), 1)
    onehot = (batch_ref[...] == gids).astype(jnp.float32)
    sums = jax.lax.dot_general(onehot, h, (((0,), (0,)), ((), ())),
                               preferred_element_type=jnp.float32)
    ones = jnp.ones((n, 1), jnp.float32)
    cnt = jax.lax.dot_general(onehot, ones, (((0,), (0,)), ((), ())),
                              preferred_element_type=jnp.float32)
    pooled = sums / jnp.maximum(cnt, 1.0)
    logits = jnp.dot(pooled, wfc_ref[...],
                     preferred_element_type=jnp.float32) + bfc_ref[...]
    m = jnp.max(logits, axis=1, keepdims=True)
    z = logits - m
    lse = jnp.log(jnp.sum(jnp.exp(z), axis=1, keepdims=True))
    out_ref[...] = z - lse


def kernel(x, edge_index, batch, W1, b1, W2, b2, Wfc, bfc):
    n, d_in = x.shape
    hid = W1.shape[1]
    c_out = Wfc.shape[1]
    e = edge_index.shape[1]
    g = 64
    chunk = 80

    w = 16
    npad = ((n + NS * 8 - 1) // (NS * 8)) * NS * 8
    nchunk = e // (NW * chunk)
    src1 = edge_index[0]
    dst1 = edge_index[1]
    dst_r = dst1.reshape(NW, nchunk, chunk)
    batch2 = batch[:, None]
    ones_rows = jnp.ones((chunk, w), jnp.float32)
    zhist = jnp.zeros((npad // NS, w), jnp.float32)
    ztile = jnp.zeros((npad // NS, hid), jnp.float32)

    deg_hist = _make_deg_hist(n, e, chunk, w, npad)
    edge_agg = _make_edge_agg(n, e, hid, chunk, npad)

    degp = deg_hist(dst_r, ones_rows, zhist)

    tc1 = pl.pallas_call(
        _tc1_body,
        out_shape=(jax.ShapeDtypeStruct((n, 1), jnp.float32),
                   jax.ShapeDtypeStruct((n, hid), jnp.float32)),
    )
    dinv, s1 = tc1(degp, x, W1)

    agg1 = edge_agg(s1, src1, dst1, ztile)

    tc2 = pl.pallas_call(
        _tc2_body,
        out_shape=jax.ShapeDtypeStruct((n, hid), jnp.float32),
    )
    s2 = tc2(agg1, s1, dinv, b1, W2)

    agg2 = edge_agg(s2, src1, dst1, ztile)

    tc3 = pl.pallas_call(
        functools.partial(_tc3_body, g=g),
        out_shape=jax.ShapeDtypeStruct((g, c_out), jnp.float32),
    )
    return tc3(agg2, s2, dinv, b2, batch2, Wfc, bfc)


# speedup vs baseline: 1.0465x; 1.0465x over previous
import functools

import jax
import jax.numpy as jnp
from jax import lax
from jax.experimental import pallas as pl
from jax.experimental.pallas import tpu as pltpu
from jax.experimental.pallas import tpu_sc as plsc

NC = 2
NS = 16
NW = NC * NS


def _make_deg_hist(n, e, chunk, w, npad):
    epw = e // NW
    nchunk = epw // chunk
    rpt = npad // NS
    mesh = plsc.VectorSubcoreMesh(core_axis_name="c", subcore_axis_name="s",
                                  num_cores=NC)

    grp = 5

    @functools.partial(
        pl.kernel,
        out_type=jax.ShapeDtypeStruct((NC, npad, w), jnp.float32),
        mesh=mesh,
        scratch_types=[
            pltpu.VMEM((nchunk, chunk), jnp.int32),
            pltpu.VMEM((chunk, w), jnp.float32),
            pltpu.VMEM_SHARED((npad, w), jnp.float32),
            pltpu.SemaphoreType.DMA,
        ],
        compiler_params=pltpu.CompilerParams(use_tc_tiling_on_sc=False),
    )
    def deg_hist(e_hbm, ones_hbm, zhist_hbm, out_hbm, dstv, rows, acc, sem):
        c = lax.axis_index("c")
        s = lax.axis_index("s")
        wid = s * NC + c
        pltpu.sync_copy(zhist_hbm, acc.at[pl.ds(s * rpt, rpt)])
        pltpu.sync_copy(ones_hbm, rows)
        pltpu.sync_copy(e_hbm.at[1, wid], dstv)
        plsc.subcore_barrier()

        def body(j0, _):
            for b in range(grp):
                pltpu.async_copy(rows, acc.at[dstv.at[j0 * grp + b]], sem,
                                 add=True)
            for b in range(grp):
                pltpu.make_async_copy(rows, acc.at[dstv.at[0]], sem).wait()
            return _

        lax.fori_loop(0, nchunk // grp, body, 0)
        plsc.subcore_barrier()
        pltpu.sync_copy(acc.at[pl.ds(s * rpt, rpt)],
                        out_hbm.at[c, pl.ds(s * rpt, rpt)])

    return deg_hist


def _make_edge_agg(n, e, d, chunk, npad):
    epw = e // NW
    nchunk = epw // chunk
    rpt = npad // NS
    mesh = plsc.VectorSubcoreMesh(core_axis_name="c", subcore_axis_name="s",
                                  num_cores=NC)
    nbuf = 4
    nbi = 8
    assert nchunk >= 10

    @functools.partial(
        pl.kernel,
        out_type=jax.ShapeDtypeStruct((NC, npad, d), jnp.float32),
        mesh=mesh,
        scratch_types=[
            [pltpu.VMEM((chunk,), jnp.int32)] * nbi,
            [pltpu.VMEM((chunk,), jnp.int32)] * nbi,
            [pltpu.VMEM((chunk, d), jnp.float32)] * nbuf,
            pltpu.VMEM_SHARED((npad, d), jnp.float32),
            [pltpu.SemaphoreType.DMA] * nbi,
            [pltpu.SemaphoreType.DMA] * nbi,
            [pltpu.SemaphoreType.DMA] * nbuf,
            [pltpu.SemaphoreType.DMA] * nbuf,
        ],
        compiler_params=pltpu.CompilerParams(use_tc_tiling_on_sc=False),
    )
    def edge_agg(s_hbm, e_hbm, ztile_hbm, out_hbm,
                 srcb, dstb, rows, acc, isem, jsem, gsem, ssem):
        c = lax.axis_index("c")
        s = lax.axis_index("s")
        wid = s * NC + c
        pltpu.sync_copy(ztile_hbm, acc.at[pl.ds(s * rpt, rpt)])
        plsc.subcore_barrier()

        def idx_start(k, bi):
            pltpu.async_copy(e_hbm.at[0, wid, k], srcb[bi], isem[bi])
            pltpu.async_copy(e_hbm.at[1, wid, k], dstb[bi], jsem[bi])

        def idx_wait(bi):
            pltpu.make_async_copy(e_hbm.at[0, 0, 0], srcb[bi],
                                  isem[bi]).wait()
            pltpu.make_async_copy(e_hbm.at[1, 0, 0], dstb[bi],
                                  jsem[bi]).wait()

        def g_start(bi, br):
            pltpu.async_copy(s_hbm.at[srcb[bi]], rows[br], gsem[br])

        def g_wait(br):
            pltpu.make_async_copy(s_hbm.at[srcb[0]], rows[br],
                                  gsem[br]).wait()

        def s_start(bij, br):
            pltpu.async_copy(rows[br], acc.at[dstb[bij]], ssem[br], add=True)

        def s_wait(br):
            pltpu.make_async_copy(rows[0], acc.at[dstb[0]], ssem[br]).wait()

        def step(j, jm4, jm8, do_sswait, do_gather, do_idx):
            g_wait(jm4)
            s_start(jm8, jm4)
            if do_sswait:
                s_wait((jm4 + 2) % nbuf)
            if do_gather:
                idx_wait((jm8 + 2) % nbi)
                g_start((jm8 + 2) % nbi, (jm4 + 2) % nbuf)
            if do_idx:
                idx_start(j + 4, (jm8 + 4) % nbi)

        for k in range(4):
            idx_start(k, k)
        idx_wait(0)
        g_start(0, 0)
        idx_wait(1)
        g_start(1, 1)
        step(0, 0, 0, False, True, True)
        step(1, 1, 1, False, True, True)

        main0 = 2
        nmain = ((min(nchunk - 4, nchunk - 2) - main0) // nbi) * nbi

        def body(j0, _):
            jb = main0 + j0 * nbi
            for i in range(nbi):
                step(jb + i, (main0 + i) % nbuf, (main0 + i) % nbi,
                     True, True, True)
            return _

        lax.fori_loop(0, nmain // nbi, body, 0)
        for j in range(main0 + nmain, nchunk):
            step(j, j % nbuf, j % nbi,
                 True, j + 2 < nchunk, j + 4 < nchunk)
        s_wait((nchunk - 2) % nbuf)
        s_wait((nchunk - 1) % nbuf)
        plsc.subcore_barrier()
        pltpu.sync_copy(acc.at[pl.ds(s * rpt, rpt)],
                        out_hbm.at[c, pl.ds(s * rpt, rpt)])

    return edge_agg


def _tc1_body(degp_ref, x_ref, w1_ref, dinv_ref, s1_ref):
    n = x_ref.shape[0]
    w = degp_ref.shape[2]
    degp = degp_ref[0, :n] + degp_ref[1, :n]
    ones = jnp.full((w, 1), 1.0 / w, jnp.float32)
    deg = jnp.dot(degp, ones, preferred_element_type=jnp.float32)
    dinv = jax.lax.rsqrt(deg + 1.0)
    dinv_ref[...] = dinv
    h = jnp.dot(x_ref[...], w1_ref[...], preferred_element_type=jnp.float32)
    s1_ref[...] = h * dinv


def _tc2_body(aggp_ref, s_ref, dinv_ref, b_ref, w2_ref, s2_ref):
    n = s_ref.shape[0]
    dinv = dinv_ref[...]
    agg = aggp_ref[0, :n] + aggp_ref[1, :n] + s_ref[...]
    h = jnp.maximum(agg * dinv + b_ref[...], 0.0)
    s2_ref[...] = jnp.dot(h, w2_ref[...],
                          preferred_element_type=jnp.float32) * dinv


def _tc3_body(aggp_ref, s_ref, dinv_ref, b_ref, batch_ref, wfc_ref, bfc_ref,
              out_ref, *, g):
    n = s_ref.shape[0]
    agg = aggp_ref[0, :n] + aggp_ref[1, :n] + s_ref[...]
    h = jnp.maximum(agg * dinv_ref[...] + b_ref[...], 0.0)
    gids = lax.broadcasted_iota(jnp.int32, (n, g), 1)
    onehot = (batch_ref[...] == gids).astype(jnp.float32)
    sums = jax.lax.dot_general(onehot, h, (((0,), (0,)), ((), ())),
                               preferred_element_type=jnp.float32)
    ones = jnp.ones((n, 1), jnp.float32)
    cnt = jax.lax.dot_general(onehot, ones, (((0,), (0,)), ((), ())),
                              preferred_element_type=jnp.float32)
    pooled = sums / jnp.maximum(cnt, 1.0)
    logits = jnp.dot(pooled, wfc_ref[...],
                     preferred_element_type=jnp.float32) + bfc_ref[...]
    m = jnp.max(logits, axis=1, keepdims=True)
    z = logits - m
    lse = jnp.log(jnp.sum(jnp.exp(z), axis=1, keepdims=True))
    out_ref[...] = z - lse


def kernel(x, edge_index, batch, W1, b1, W2, b2, Wfc, bfc):
    n, d_in = x.shape
    hid = W1.shape[1]
    c_out = Wfc.shape[1]
    e = edge_index.shape[1]
    g = 64
    chunk = 80

    w = 16
    npad = ((n + NS * 8 - 1) // (NS * 8)) * NS * 8
    nchunk = e // (NW * chunk)
    edge3 = edge_index.reshape(2, NW, nchunk, chunk)
    batch2 = batch[:, None]
    ones_rows = jnp.ones((chunk, w), jnp.float32)
    zhist = jnp.zeros((npad // NS, w), jnp.float32)
    ztile = jnp.zeros((npad // NS, hid), jnp.float32)

    deg_hist = _make_deg_hist(n, e, chunk, w, npad)
    edge_agg = _make_edge_agg(n, e, hid, chunk, npad)

    degp = deg_hist(edge3, ones_rows, zhist)

    tc1 = pl.pallas_call(
        _tc1_body,
        out_shape=(jax.ShapeDtypeStruct((n, 1), jnp.float32),
                   jax.ShapeDtypeStruct((n, hid), jnp.float32)),
    )
    dinv, s1 = tc1(degp, x, W1)

    agg1 = edge_agg(s1, edge3, ztile)

    tc2 = pl.pallas_call(
        _tc2_body,
        out_shape=jax.ShapeDtypeStruct((n, hid), jnp.float32),
    )
    s2 = tc2(agg1, s1, dinv, b1, W2)

    agg2 = edge_agg(s2, edge3, ztile)

    tc3 = pl.pallas_call(
        functools.partial(_tc3_body, g=g),
        out_shape=jax.ShapeDtypeStruct((g, c_out), jnp.float32),
    )
    return tc3(agg2, s2, dinv, b2, batch2, Wfc, bfc)


# speedup vs baseline: 1.0518x; 1.0050x over previous
import functools

import jax
import jax.numpy as jnp
from jax import lax
from jax.experimental import pallas as pl
from jax.experimental.pallas import tpu as pltpu
from jax.experimental.pallas import tpu_sc as plsc

NC = 2
NS = 16
NW = NC * NS


def _make_deg_hist(n, e, chunk, w, npad):
    epw = e // NW
    nchunk = epw // chunk
    rpt = npad // NS
    mesh = plsc.VectorSubcoreMesh(core_axis_name="c", subcore_axis_name="s",
                                  num_cores=NC)

    grp = 5

    @functools.partial(
        pl.kernel,
        out_type=jax.ShapeDtypeStruct((NC, npad, w), jnp.float32),
        mesh=mesh,
        scratch_types=[
            pltpu.VMEM((nchunk, chunk), jnp.int32),
            pltpu.VMEM((chunk, w), jnp.float32),
            pltpu.VMEM_SHARED((npad, w), jnp.float32),
            pltpu.SemaphoreType.DMA,
        ],
        compiler_params=pltpu.CompilerParams(use_tc_tiling_on_sc=False),
    )
    def deg_hist(e_hbm, ones_hbm, zhist_hbm, out_hbm, dstv, rows, acc, sem):
        c = lax.axis_index("c")
        s = lax.axis_index("s")
        wid = s * NC + c
        pltpu.sync_copy(zhist_hbm, acc.at[pl.ds(s * rpt, rpt)])
        pltpu.sync_copy(ones_hbm, rows)
        pltpu.sync_copy(e_hbm.at[1, wid], dstv)
        plsc.subcore_barrier()

        def body(j0, _):
            for b in range(grp):
                pltpu.async_copy(rows, acc.at[dstv.at[j0 * grp + b]], sem,
                                 add=True)
            for b in range(grp):
                pltpu.make_async_copy(rows, acc.at[dstv.at[0]], sem).wait()
            return _

        lax.fori_loop(0, nchunk // grp, body, 0)
        plsc.subcore_barrier()
        pltpu.sync_copy(acc.at[pl.ds(s * rpt, rpt)],
                        out_hbm.at[c, pl.ds(s * rpt, rpt)])

    return deg_hist


def _make_edge_agg(n, e, d, chunk, npad):
    epw = e // NW
    nchunk = epw // chunk
    rpt = npad // NS
    mesh = plsc.VectorSubcoreMesh(core_axis_name="c", subcore_axis_name="s",
                                  num_cores=NC)
    nbuf = 4
    nbi = 8
    assert nchunk >= 10

    @functools.partial(
        pl.kernel,
        out_type=jax.ShapeDtypeStruct((NC, npad, d), jnp.float32),
        mesh=mesh,
        scratch_types=[
            [pltpu.VMEM((chunk,), jnp.int32)] * nbi,
            [pltpu.VMEM((chunk,), jnp.int32)] * nbi,
            [pltpu.VMEM((chunk, d), jnp.float32)] * nbuf,
            pltpu.VMEM_SHARED((npad, d), jnp.float32),
            [pltpu.SemaphoreType.DMA] * nbi,
            [pltpu.SemaphoreType.DMA] * nbi,
            [pltpu.SemaphoreType.DMA] * nbuf,
            [pltpu.SemaphoreType.DMA] * nbuf,
        ],
        compiler_params=pltpu.CompilerParams(use_tc_tiling_on_sc=False),
    )
    def edge_agg(s_hbm, e_hbm, ztile_hbm, out_hbm,
                 srcb, dstb, rows, acc, isem, jsem, gsem, ssem):
        c = lax.axis_index("c")
        s = lax.axis_index("s")
        wid = s * NC + c

        def idx_start(k, bi):
            pltpu.async_copy(e_hbm.at[0, wid, k], srcb[bi], isem[bi])
            pltpu.async_copy(e_hbm.at[1, wid, k], dstb[bi], jsem[bi])

        def idx_wait(bi):
            pltpu.make_async_copy(e_hbm.at[0, 0, 0], srcb[bi],
                                  isem[bi]).wait()
            pltpu.make_async_copy(e_hbm.at[1, 0, 0], dstb[bi],
                                  jsem[bi]).wait()

        def g_start(bi, br):
            pltpu.async_copy(s_hbm.at[srcb[bi]], rows[br], gsem[br])

        def g_wait(br):
            pltpu.make_async_copy(s_hbm.at[srcb[0]], rows[br],
                                  gsem[br]).wait()

        def s_start(bij, br):
            pltpu.async_copy(rows[br], acc.at[dstb[bij]], ssem[br], add=True)

        def s_wait(br):
            pltpu.make_async_copy(rows[0], acc.at[dstb[0]], ssem[br]).wait()

        def step(j, jm4, jm8, do_sswait, do_gather, do_idx):
            g_wait(jm4)
            s_start(jm8, jm4)
            if do_idx:
                idx_start(j + 4, (jm8 + 4) % nbi)
            if do_sswait:
                s_wait((jm4 + 2) % nbuf)
            if do_gather:
                idx_wait((jm8 + 2) % nbi)
                g_start((jm8 + 2) % nbi, (jm4 + 2) % nbuf)

        for k in range(4):
            idx_start(k, k)
        pltpu.sync_copy(ztile_hbm, acc.at[pl.ds(s * rpt, rpt)])
        idx_wait(0)
        g_start(0, 0)
        idx_wait(1)
        g_start(1, 1)
        plsc.subcore_barrier()
        step(0, 0, 0, False, True, True)
        step(1, 1, 1, False, True, True)

        main0 = 2
        nmain = ((min(nchunk - 4, nchunk - 2) - main0) // nbi) * nbi

        def body(j0, _):
            jb = main0 + j0 * nbi
            for i in range(nbi):
                step(jb + i, (main0 + i) % nbuf, (main0 + i) % nbi,
                     True, True, True)
            return _

        lax.fori_loop(0, nmain // nbi, body, 0)
        for j in range(main0 + nmain, nchunk):
            step(j, j % nbuf, j % nbi,
                 True, j + 2 < nchunk, j + 4 < nchunk)
        s_wait((nchunk - 2) % nbuf)
        s_wait((nchunk - 1) % nbuf)
        plsc.subcore_barrier()
        pltpu.sync_copy(acc.at[pl.ds(s * rpt, rpt)],
                        out_hbm.at[c, pl.ds(s * rpt, rpt)])

    return edge_agg


def _tc1_body(degp_ref, x_ref, w1_ref, dinv_ref, s1_ref):
    n = x_ref.shape[0]
    w = degp_ref.shape[2]
    degp = degp_ref[0, :n] + degp_ref[1, :n]
    ones = jnp.full((w, 1), 1.0 / w, jnp.float32)
    deg = jnp.dot(degp, ones, preferred_element_type=jnp.float32)
    dinv = jax.lax.rsqrt(deg + 1.0)
    dinv_ref[...] = dinv
    h = jnp.dot(x_ref[...], w1_ref[...], preferred_element_type=jnp.float32)
    s1_ref[...] = h * dinv


def _tc2_body(aggp_ref, s_ref, dinv_ref, b_ref, w2_ref, s2_ref):
    n = s_ref.shape[0]
    dinv = dinv_ref[...]
    agg = aggp_ref[0, :n] + aggp_ref[1, :n] + s_ref[...]
    h = jnp.maximum(agg * dinv + b_ref[...], 0.0)
    s2_ref[...] = jnp.dot(h, w2_ref[...],
                          preferred_element_type=jnp.float32) * dinv


def _tc3_body(aggp_ref, s_ref, dinv_ref, b_ref, batch_ref, wfc_ref, bfc_ref,
              out_ref, *, g):
    n = s_ref.shape[0]
    agg = aggp_ref[0, :n] + aggp_ref[1, :n] + s_ref[...]
    h = jnp.maximum(agg * dinv_ref[...] + b_ref[...], 0.0)
    gids = lax.broadcasted_iota(jnp.int32, (n, g), 1)
    onehot = (batch_ref[...] == gids).astype(jnp.float32)
    sums = jax.lax.dot_general(onehot, h, (((0,), (0,)), ((), ())),
                               preferred_element_type=jnp.float32)
    ones = jnp.ones((n, 1), jnp.float32)
    cnt = jax.lax.dot_general(onehot, ones, (((0,), (0,)), ((), ())),
                              preferred_element_type=jnp.float32)
    pooled = sums / jnp.maximum(cnt, 1.0)
    logits = jnp.dot(pooled, wfc_ref[...],
                     preferred_element_type=jnp.float32) + bfc_ref[...]
    m = jnp.max(logits, axis=1, keepdims=True)
    z = logits - m
    lse = jnp.log(jnp.sum(jnp.exp(z), axis=1, keepdims=True))
    out_ref[...] = z - lse


def kernel(x, edge_index, batch, W1, b1, W2, b2, Wfc, bfc):
    n, d_in = x.shape
    hid = W1.shape[1]
    c_out = Wfc.shape[1]
    e = edge_index.shape[1]
    g = 64
    chunk = 80

    w = 16
    npad = ((n + NS * 8 - 1) // (NS * 8)) * NS * 8
    nchunk = e // (NW * chunk)
    edge3 = edge_index.reshape(2, NW, nchunk, chunk)
    batch2 = batch[:, None]
    ones_rows = jnp.ones((chunk, w), jnp.float32)
    zhist = jnp.zeros((npad // NS, w), jnp.float32)
    ztile = jnp.zeros((npad // NS, hid), jnp.float32)

    deg_hist = _make_deg_hist(n, e, chunk, w, npad)
    edge_agg = _make_edge_agg(n, e, hid, chunk, npad)

    degp = deg_hist(edge3, ones_rows, zhist)

    tc1 = pl.pallas_call(
        _tc1_body,
        out_shape=(jax.ShapeDtypeStruct((n, 1), jnp.float32),
                   jax.ShapeDtypeStruct((n, hid), jnp.float32)),
    )
    dinv, s1 = tc1(degp, x, W1)

    agg1 = edge_agg(s1, edge3, ztile)

    tc2 = pl.pallas_call(
        _tc2_body,
        out_shape=jax.ShapeDtypeStruct((n, hid), jnp.float32),
    )
    s2 = tc2(agg1, s1, dinv, b1, W2)

    agg2 = edge_agg(s2, edge3, ztile)

    tc3 = pl.pallas_call(
        functools.partial(_tc3_body, g=g),
        out_shape=jax.ShapeDtypeStruct((g, c_out), jnp.float32),
    )
    return tc3(agg2, s2, dinv, b2, batch2, Wfc, bfc)
